# Initial kernel scaffold; baseline (speedup 1.0000x reference)
import jax
import jax.numpy as jnp
from jax.experimental import pallas as pl


def kernel(entity_reps, restriction_maps, edge_index, entity_types):
    raise NotImplementedError("write your pallas kernel here")


# speedup vs baseline: 156.0860x; 156.0860x over previous
import functools

import jax
import jax.numpy as jnp
from jax import lax
from jax.experimental import pallas as pl
from jax.experimental.pallas import tpu as pltpu
from jax.experimental.pallas import tpu_sc as plsc

N_NODES = 10000
N_EDGES = 320000
D = 8
T = 8
HN = 10240
HR = 9
HSIZE = HR * HN
DEG_BASE = T * HN

NC = 2
NS = 16
NW = NC * NS
EPW = N_EDGES // NW
L = 16


def _sc_hist_body(row_hbm, col_hbm, types_hbm, out_hbm, hist, types_v, rowb, colb):
    cid = lax.axis_index("c")
    sid = lax.axis_index("s")
    wid = sid * NC + cid

    zf = jnp.zeros((L,), jnp.float32)

    def zero_body(i, _):
        base = i * (L * 16)
        for k in range(16):
            hist[pl.ds(base + k * L, L)] = zf
        return 0

    lax.fori_loop(0, HSIZE // (L * 16), zero_body, 0)

    pltpu.sync_copy(types_hbm, types_v)
    ebase = wid * EPW
    pltpu.sync_copy(row_hbm.at[pl.ds(ebase, EPW)], rowb)
    pltpu.sync_copy(col_hbm.at[pl.ds(ebase, EPW)], colb)

    ones = jnp.ones((L,), jnp.float32)
    neg_ones = -ones

    def group(g):
        r = rowb[pl.ds(g * L, L)]
        c = colb[pl.ds(g * L, L)]
        t_r = plsc.load_gather(types_v, [r])
        t_c = plsc.load_gather(types_v, [c])
        plsc.addupdate_scatter(hist, [t_c * HN + r], ones)
        plsc.addupdate_scatter(hist, [t_r * HN + c], neg_ones)
        plsc.addupdate_scatter(hist, [c + DEG_BASE], ones)

    def edge_body(i, _):
        for k in range(5):
            group(i * 5 + k)
        return 0

    lax.fori_loop(0, EPW // (L * 5), edge_body, 0)

    pltpu.sync_copy(hist, out_hbm.at[wid])


@functools.cache
def _sc_hist():
    return functools.partial(
        pl.kernel,
        out_type=jax.ShapeDtypeStruct((NW, HSIZE), jnp.float32),
        mesh=plsc.VectorSubcoreMesh(
            core_axis_name="c", subcore_axis_name="s",
            num_cores=NC, num_subcores=NS,
        ),
        scratch_types=[
            pltpu.VMEM((HSIZE,), jnp.float32),
            pltpu.VMEM((HN,), jnp.int32),
            pltpu.VMEM((EPW,), jnp.int32),
            pltpu.VMEM((EPW,), jnp.int32),
        ],
        compiler_params=pltpu.CompilerParams(needs_layout_passes=False),
    )(_sc_hist_body)


def _tc_final_body(hists_ref, e_ref, l_ref, ty_ref, out_ref):
    s = hists_ref[0]
    for w in range(1, NW):
        s = s + hists_ref[w]
    cnt = s[:T, :]
    deg = s[T:T + 1, :]

    b = jnp.zeros((D * D, HN), jnp.float32)
    for t in range(T):
        d = jnp.dot(l_ref[t], cnt, preferred_element_type=jnp.float32)
        m = ty_ref[...] == t
        b = b + jnp.where(m, d, 0.0)

    acc = jnp.zeros((D, HN), jnp.float32)
    for j in range(D):
        acc = acc + b[j * D:(j + 1) * D, :] * e_ref[j:j + 1, :]

    deg_inv = jnp.where(deg > 0, 1.0 / deg, 0.0)
    out_ref[0, 0] = jnp.sum(acc * acc * deg_inv)


_tc_final = pl.pallas_call(
    _tc_final_body,
    out_shape=jax.ShapeDtypeStruct((1, 1), jnp.float32),
    out_specs=pl.BlockSpec(memory_space=pltpu.SMEM),
)


def kernel(entity_reps, restriction_maps, edge_index, entity_types):
    types_pad = jnp.zeros((HN,), jnp.int32).at[:N_NODES].set(entity_types)
    e_pad = jnp.zeros((D, HN), jnp.float32).at[:, :N_NODES].set(entity_reps)
    l_maps = jnp.transpose(restriction_maps, (0, 3, 2, 1)).reshape(T, D * D, T)

    hists = _sc_hist()(edge_index[0], edge_index[1], types_pad)
    hists = hists.reshape(NW, HR, HN)
    out = _tc_final(hists, e_pad, l_maps, types_pad.reshape(1, HN))
    return out[0, 0]


# speedup vs baseline: 216.7090x; 1.3884x over previous
import functools

import jax
import jax.numpy as jnp
from jax import lax
from jax.experimental import pallas as pl
from jax.experimental.pallas import tpu as pltpu
from jax.experimental.pallas import tpu_sc as plsc

N_NODES = 10000
N_EDGES = 320000
D = 8
T = 8
HN = 10240
HR = 9
HSIZE = HR * HN
DEG_BASE = T * HN

NC = 2
NS = 16
NW = NC * NS
EPW = N_EDGES // NW
L = 16


def _sc_hist_body(row_hbm, col_hbm, types_hbm, out_hbm, hist, types_v, rowb, colb):
    cid = lax.axis_index("c")
    sid = lax.axis_index("s")
    wid = sid * NC + cid

    zf = jnp.zeros((L,), jnp.float32)

    def zero_body(i, _):
        base = i * (L * 16)
        for k in range(16):
            hist[pl.ds(base + k * L, L)] = zf
        return 0

    lax.fori_loop(0, HSIZE // (L * 16), zero_body, 0)

    pltpu.sync_copy(types_hbm, types_v)
    ebase = wid * EPW
    pltpu.sync_copy(row_hbm.at[pl.ds(ebase, EPW)], rowb)
    pltpu.sync_copy(col_hbm.at[pl.ds(ebase, EPW)], colb)

    ones = jnp.ones((L,), jnp.float32)
    neg_ones = -ones

    def group(g):
        r = rowb[pl.ds(g * L, L)]
        c = colb[pl.ds(g * L, L)]
        t_r = plsc.load_gather(types_v, [r])
        t_c = plsc.load_gather(types_v, [c])
        plsc.addupdate_scatter(hist, [t_c * HN + r], ones)
        plsc.addupdate_scatter(hist, [t_r * HN + c], neg_ones)
        plsc.addupdate_scatter(hist, [c + DEG_BASE], ones)

    def edge_body(i, _):
        for k in range(5):
            group(i * 5 + k)
        return 0

    lax.fori_loop(0, EPW // (L * 5), edge_body, 0)

    pltpu.sync_copy(hist, out_hbm.at[wid])


@functools.cache
def _sc_hist():
    return functools.partial(
        pl.kernel,
        out_type=jax.ShapeDtypeStruct((NW, HSIZE), jnp.float32),
        mesh=plsc.VectorSubcoreMesh(
            core_axis_name="c", subcore_axis_name="s",
            num_cores=NC, num_subcores=NS,
        ),
        scratch_types=[
            pltpu.VMEM((HSIZE,), jnp.float32),
            pltpu.VMEM((HN,), jnp.int32),
            pltpu.VMEM((EPW,), jnp.int32),
            pltpu.VMEM((EPW,), jnp.int32),
        ],
        compiler_params=pltpu.CompilerParams(needs_layout_passes=False),
    )(_sc_hist_body)


def _tc_final_body(hists_ref, e_ref, l_ref, ty_ref, out_ref):
    ones_l = jnp.ones((8, NW), jnp.float32)
    s8 = jnp.dot(ones_l, hists_ref[...], preferred_element_type=jnp.float32)
    cnt = jnp.concatenate(
        [s8[u:u + 1, u * HN:(u + 1) * HN] for u in range(T)], axis=0
    )
    deg = s8[0:1, DEG_BASE:DEG_BASE + HN]

    b = jnp.zeros((D * D, HN), jnp.float32)
    for t in range(T):
        d = jnp.dot(l_ref[t], cnt, preferred_element_type=jnp.float32)
        m = ty_ref[...] == t
        b = b + jnp.where(m, d, 0.0)

    acc = jnp.zeros((D, HN), jnp.float32)
    for j in range(D):
        acc = acc + b[j * D:(j + 1) * D, :] * e_ref[j:j + 1, :]

    deg_inv = jnp.where(deg > 0, 1.0 / deg, 0.0)
    out_ref[0, 0] = jnp.sum(acc * acc * deg_inv)


_tc_final = pl.pallas_call(
    _tc_final_body,
    out_shape=jax.ShapeDtypeStruct((1, 1), jnp.float32),
    out_specs=pl.BlockSpec(memory_space=pltpu.SMEM),
)


def kernel(entity_reps, restriction_maps, edge_index, entity_types):
    types_pad = jnp.zeros((HN,), jnp.int32).at[:N_NODES].set(entity_types)
    e_pad = jnp.zeros((D, HN), jnp.float32).at[:, :N_NODES].set(entity_reps)
    l_maps = jnp.transpose(restriction_maps, (0, 3, 2, 1)).reshape(T, D * D, T)

    hists = _sc_hist()(edge_index[0], edge_index[1], types_pad)
    out = _tc_final(hists, e_pad, l_maps, types_pad.reshape(1, HN))
    return out[0, 0]


# speedup vs baseline: 296.6477x; 1.3689x over previous
import functools

import jax
import jax.numpy as jnp
from jax import lax
from jax.experimental import pallas as pl
from jax.experimental.pallas import tpu as pltpu
from jax.experimental.pallas import tpu_sc as plsc

N_NODES = 10000
N_EDGES = 320000
D = 8
T = 8
HN = 10240
HR = 9
HSIZE = HR * HN
DEG_BASE = T * HN

NC = 2
NS = 16
NW = NC * NS
EPW = N_EDGES // NW
L = 16


EWIN = EPW + 112
CHUNKS = [1280] * 7 + [1152]
CMAX = 1280


def _sc_hist_body(edge_hbm, types_hbm, out_hbm, hist, types_v, ebuf0, ebuf1,
                  sem_t, sem0, sem1):
    cid = lax.axis_index("c")
    sid = lax.axis_index("s")
    wid = sid * NC + cid

    ebase = wid * EPW
    wstart = pl.multiple_of((ebase // 128) * 128, 128)
    off0 = ebase - wstart

    tcopy = pltpu.async_copy(types_hbm, types_v, sem_t)
    ebufs = [ebuf0, ebuf1]
    sems = [sem0, sem1]
    cstarts = [sum(CHUNKS[:k]) for k in range(len(CHUNKS))]
    copies = [None] * len(CHUNKS)
    copies[0] = pltpu.async_copy(
        edge_hbm.at[:, pl.ds(wstart, CHUNKS[0])], ebufs[0], sems[0]
    )

    zf = jnp.zeros((L,), jnp.float32)

    def zero_body(i, _):
        base = i * (L * 16)
        for k in range(16):
            hist[pl.ds(base + k * L, L)] = zf
        return 0

    lax.fori_loop(0, HSIZE // (L * 16), zero_body, 0)
    tcopy.wait()

    ones = jnp.ones((L,), jnp.float32)
    neg_ones = -ones
    zeros_i = jnp.zeros((L,), jnp.int32)
    ones_i = jnp.ones((L,), jnp.int32)
    lane_iota = lax.iota(jnp.int32, L)

    for k, clen in enumerate(CHUNKS):
        nxt = k + 1
        if nxt < len(CHUNKS):
            copies[nxt] = pltpu.async_copy(
                edge_hbm.at[:, pl.ds(wstart + cstarts[nxt], CHUNKS[nxt])],
                ebufs[nxt % 2] if CHUNKS[nxt] == CMAX
                else ebufs[nxt % 2].at[:, pl.ds(0, CHUNKS[nxt])],
                sems[nxt % 2],
            )
        copies[k].wait()
        ebuf = ebufs[k % 2]

        cs, ce = cstarts[k], cstarts[k] + clen
        lo = jnp.maximum(off0, cs)
        hi = jnp.minimum(off0 + EPW, ce)

        def group_body(g, _, lo=lo, cs=cs, ebuf=ebuf):
            idx = (lo - cs + g * L) + lane_iota
            r = plsc.load_gather(ebuf, [zeros_i, idx])
            c = plsc.load_gather(ebuf, [ones_i, idx])
            t_r = plsc.load_gather(types_v, [r])
            t_c = plsc.load_gather(types_v, [c])
            plsc.addupdate_scatter(hist, [t_c * HN + r], ones)
            plsc.addupdate_scatter(hist, [t_r * HN + c], neg_ones)
            plsc.addupdate_scatter(hist, [c + DEG_BASE], ones)
            return 0

        lax.fori_loop(0, (hi - lo) // L, group_body, 0)

    pltpu.sync_copy(hist, out_hbm.at[wid])


@functools.cache
def _sc_hist():
    return functools.partial(
        pl.kernel,
        out_type=jax.ShapeDtypeStruct((NW, HSIZE), jnp.float32),
        mesh=plsc.VectorSubcoreMesh(
            core_axis_name="c", subcore_axis_name="s",
            num_cores=NC, num_subcores=NS,
        ),
        scratch_types=[
            pltpu.VMEM((HSIZE,), jnp.float32),
            pltpu.VMEM((HN,), jnp.int32),
            pltpu.VMEM((2, CMAX), jnp.int32),
            pltpu.VMEM((2, CMAX), jnp.int32),
            pltpu.SemaphoreType.DMA,
            pltpu.SemaphoreType.DMA,
            pltpu.SemaphoreType.DMA,
        ],
        compiler_params=pltpu.CompilerParams(needs_layout_passes=False),
    )(_sc_hist_body)


def _tc_final_body(hists_ref, e_ref, l_ref, ty_ref, out_ref):
    ones_l = jnp.ones((8, NW), jnp.float32)
    s8 = jnp.dot(ones_l, hists_ref[...], preferred_element_type=jnp.float32)
    cnt = jnp.concatenate(
        [s8[u:u + 1, u * HN:(u + 1) * HN] for u in range(T)], axis=0
    )
    deg = s8[0:1, DEG_BASE:DEG_BASE + HN]

    b = jnp.zeros((D * D, HN), jnp.float32)
    for t in range(T):
        d = jnp.dot(l_ref[t], cnt, preferred_element_type=jnp.float32)
        m = ty_ref[...] == t
        b = b + jnp.where(m, d, 0.0)

    acc = jnp.zeros((D, HN), jnp.float32)
    for j in range(D):
        acc = acc + b[j * D:(j + 1) * D, :] * e_ref[j:j + 1, :]

    deg_inv = jnp.where(deg > 0, 1.0 / deg, 0.0)
    out_ref[0, 0] = jnp.sum(acc * acc * deg_inv)


_tc_final = pl.pallas_call(
    _tc_final_body,
    out_shape=jax.ShapeDtypeStruct((1, 1), jnp.float32),
    out_specs=pl.BlockSpec(memory_space=pltpu.SMEM),
)


def kernel(entity_reps, restriction_maps, edge_index, entity_types):
    types_pad = jnp.zeros((HN,), jnp.int32).at[:N_NODES].set(entity_types)
    e_pad = jnp.zeros((D, HN), jnp.float32).at[:, :N_NODES].set(entity_reps)
    l_maps = jnp.transpose(restriction_maps, (0, 3, 2, 1)).reshape(T, D * D, T)

    hists = _sc_hist()(edge_index, types_pad)
    out = _tc_final(hists, e_pad, l_maps, types_pad.reshape(1, HN))
    return out[0, 0]


# speedup vs baseline: 328.1192x; 1.1061x over previous
import functools

import jax
import jax.numpy as jnp
from jax import lax
from jax.experimental import pallas as pl
from jax.experimental.pallas import tpu as pltpu
from jax.experimental.pallas import tpu_sc as plsc

N_NODES = 10000
N_EDGES = 320000
D = 8
T = 8
HN = 10240
HR = 9
HSIZE = HR * HN
DEG_BASE = T * HN

NC = 2
NS = 16
NW = NC * NS
EPW = N_EDGES // NW
L = 16


EWIN = EPW + 112
CHUNKS = [1280] * 7 + [1152]
CMAX = 1280


def _sc_hist_body(edge_hbm, types_hbm, out_hbm, hist, types_v, ebuf0, ebuf1,
                  sem_t, sem0, sem1):
    cid = lax.axis_index("c")
    sid = lax.axis_index("s")
    wid = sid * NC + cid

    ebase = wid * EPW
    wstart = pl.multiple_of((ebase // 128) * 128, 128)
    off0 = ebase - wstart

    tcopy = pltpu.async_copy(types_hbm, types_v, sem_t)
    ebufs = [ebuf0, ebuf1]
    sems = [sem0, sem1]
    cstarts = [sum(CHUNKS[:k]) for k in range(len(CHUNKS))]
    copies = [None] * len(CHUNKS)
    copies[0] = pltpu.async_copy(
        edge_hbm.at[:, pl.ds(wstart, CHUNKS[0])], ebufs[0], sems[0]
    )

    zf = jnp.zeros((L,), jnp.float32)

    def zero_body(i, _):
        base = i * (L * 16)
        for k in range(16):
            hist[pl.ds(base + k * L, L)] = zf
        return 0

    lax.fori_loop(0, HSIZE // (L * 16), zero_body, 0)
    tcopy.wait()

    ones = jnp.ones((L,), jnp.float32)
    neg_ones = -ones
    zeros_i = jnp.zeros((L,), jnp.int32)
    ones_i = jnp.ones((L,), jnp.int32)
    lane_iota = lax.iota(jnp.int32, L)

    for k, clen in enumerate(CHUNKS):
        nxt = k + 1
        if nxt < len(CHUNKS):
            copies[nxt] = pltpu.async_copy(
                edge_hbm.at[:, pl.ds(wstart + cstarts[nxt], CHUNKS[nxt])],
                ebufs[nxt % 2] if CHUNKS[nxt] == CMAX
                else ebufs[nxt % 2].at[:, pl.ds(0, CHUNKS[nxt])],
                sems[nxt % 2],
            )
        copies[k].wait()
        ebuf = ebufs[k % 2]

        cs, ce = cstarts[k], cstarts[k] + clen
        lo = jnp.maximum(off0, cs)
        hi = jnp.minimum(off0 + EPW, ce)

        @plsc.parallel_loop(0, (hi - lo) // L, unroll=5)
        def _(g, lo=lo, cs=cs, ebuf=ebuf):
            idx = (lo - cs + g * L) + lane_iota
            r = plsc.load_gather(ebuf, [zeros_i, idx])
            c = plsc.load_gather(ebuf, [ones_i, idx])
            t_r = plsc.load_gather(types_v, [r])
            t_c = plsc.load_gather(types_v, [c])
            plsc.addupdate_scatter(hist, [t_c * HN + r], ones)
            plsc.addupdate_scatter(hist, [t_r * HN + c], neg_ones)
            plsc.addupdate_scatter(hist, [c + DEG_BASE], ones)

    pltpu.sync_copy(hist, out_hbm.at[wid])


@functools.cache
def _sc_hist():
    return functools.partial(
        pl.kernel,
        out_type=jax.ShapeDtypeStruct((NW, HSIZE), jnp.float32),
        mesh=plsc.VectorSubcoreMesh(
            core_axis_name="c", subcore_axis_name="s",
            num_cores=NC, num_subcores=NS,
        ),
        scratch_types=[
            pltpu.VMEM((HSIZE,), jnp.float32),
            pltpu.VMEM((HN,), jnp.int32),
            pltpu.VMEM((2, CMAX), jnp.int32),
            pltpu.VMEM((2, CMAX), jnp.int32),
            pltpu.SemaphoreType.DMA,
            pltpu.SemaphoreType.DMA,
            pltpu.SemaphoreType.DMA,
        ],
        compiler_params=pltpu.CompilerParams(needs_layout_passes=False),
    )(_sc_hist_body)


def _tc_final_body(hists_ref, e_ref, l_ref, ty_ref, out_ref):
    ones_l = jnp.ones((8, NW), jnp.float32)
    s8 = jnp.dot(ones_l, hists_ref[...], preferred_element_type=jnp.float32)
    cnt = jnp.concatenate(
        [s8[u:u + 1, u * HN:(u + 1) * HN] for u in range(T)], axis=0
    )
    deg = s8[0:1, DEG_BASE:DEG_BASE + HN]

    b = jnp.zeros((D * D, HN), jnp.float32)
    for t in range(T):
        d = jnp.dot(l_ref[t], cnt, preferred_element_type=jnp.float32)
        m = ty_ref[...] == t
        b = b + jnp.where(m, d, 0.0)

    acc = jnp.zeros((D, HN), jnp.float32)
    for j in range(D):
        acc = acc + b[j * D:(j + 1) * D, :] * e_ref[j:j + 1, :]

    deg_inv = jnp.where(deg > 0, 1.0 / deg, 0.0)
    out_ref[0, 0] = jnp.sum(acc * acc * deg_inv)


_tc_final = pl.pallas_call(
    _tc_final_body,
    out_shape=jax.ShapeDtypeStruct((1, 1), jnp.float32),
    out_specs=pl.BlockSpec(memory_space=pltpu.SMEM),
)


def kernel(entity_reps, restriction_maps, edge_index, entity_types):
    types_pad = jnp.zeros((HN,), jnp.int32).at[:N_NODES].set(entity_types)
    e_pad = jnp.zeros((D, HN), jnp.float32).at[:, :N_NODES].set(entity_reps)
    l_maps = jnp.transpose(restriction_maps, (0, 3, 2, 1)).reshape(T, D * D, T)

    hists = _sc_hist()(edge_index, types_pad)
    out = _tc_final(hists, e_pad, l_maps, types_pad.reshape(1, HN))
    return out[0, 0]
